# baseline (speedup 1.0000x reference)
import functools

import jax
import jax.numpy as jnp
from jax import lax
from jax.experimental import pallas as pl
from jax.experimental.pallas import tpu as pltpu
from jax.experimental.pallas import tpu_sc as plsc

_N_NODES = 10000
_N_EDGES = 320000
_HIDDEN = 128
_HW = _HIDDEN // 2
_N_REL = 100
_N_SEC = 50

_NC = 2
_NS = 16
_NW = _NC * _NS
_L = 16

_EPW = _N_EDGES // _NW
_C = 80
_NCHUNK = _EPW // _C


def _pack2d(x):
    b = x.astype(jnp.bfloat16)
    lo = lax.bitcast_convert_type(b[:, :_HW], jnp.uint16).astype(jnp.uint32)
    hi = lax.bitcast_convert_type(b[:, _HW:], jnp.uint16).astype(jnp.uint32)
    return lax.bitcast_convert_type(lo | (hi << 16), jnp.int32)


def _prep_body(node_ref, etype_ref, typew_ref, relw_ref, secw_ref,
               aug_ref, relp_ref, secp_ref):
    et = etype_ref[...]
    k = lax.broadcasted_iota(jnp.int32, (et.shape[0], typew_ref.shape[0]), 1)
    onehot = (et == k).astype(jnp.float32)
    aug_ref[...] = _pack2d(node_ref[...] + jnp.dot(
        onehot, typew_ref[...], preferred_element_type=jnp.float32))
    relp_ref[...] = _pack2d(relw_ref[...])
    secp_ref[...] = _pack2d(secw_ref[...])


def _edge_body(a_hbm, head_hbm, tail_hbm, rel_hbm, sec_hbm, relw_hbm,
               secw_hbm, out_hbm, head_v, tail_v, rel_v, sec_v,
               hbuf0, hbuf1, tbuf0, tbuf1, relt, sect, out_v, sem0, sem1):
    wid = lax.axis_index("s") * _NC + lax.axis_index("c")
    base = wid * _EPW
    pltpu.sync_copy(head_hbm.at[pl.ds(base, _EPW)], head_v)
    pltpu.sync_copy(tail_hbm.at[pl.ds(base, _EPW)], tail_v)
    pltpu.sync_copy(rel_hbm.at[pl.ds(base, _EPW)], rel_v)
    pltpu.sync_copy(sec_hbm.at[pl.ds(base, _EPW)], sec_v)
    pltpu.sync_copy(relw_hbm, relt)
    pltpu.sync_copy(secw_hbm, sect)

    bufs = ((hbuf0, tbuf0, sem0), (hbuf1, tbuf1, sem1))

    def issue(c, slot):
        hb, tb, sm = bufs[slot]
        off = c * _C
        pltpu.async_copy(a_hbm.at[head_v.at[pl.ds(off, _C)]], hb, sm)
        pltpu.async_copy(a_hbm.at[tail_v.at[pl.ds(off, _C)]], tb, sm)

    def drain(c, slot):
        hb, tb, sm = bufs[slot]
        off = c * _C
        pltpu.make_async_copy(a_hbm.at[head_v.at[pl.ds(off, _C)]],
                              hb, sm).wait()
        pltpu.make_async_copy(a_hbm.at[tail_v.at[pl.ds(off, _C)]],
                              tb, sm).wait()

    def compute(c, slot):
        hb2, tb2, _ = bufs[slot]
        off = c * _C

        def blk_body(j, carry2):
            erow = j * _L + lax.iota(jnp.int32, _L)
            relids = rel_v[pl.ds(off + j * _L, _L)]
            secids = sec_v[pl.ds(off + j * _L, _L)]
            zf = jnp.zeros((_L,), jnp.float32)
            col = lax.iota(jnp.int32, _L) * 0

            def prod(cv):
                h = plsc.bitcast(plsc.load_gather(hb2, [erow, cv]),
                                 jnp.bfloat16)
                t = plsc.bitcast(plsc.load_gather(tb2, [erow, cv]),
                                 jnp.bfloat16)
                r = plsc.bitcast(plsc.load_gather(relt, [relids, cv]),
                                 jnp.bfloat16)
                s = plsc.bitcast(plsc.load_gather(sect, [secids, cv]),
                                 jnp.bfloat16)
                return h * t * (r + s)

            def w_body(k, carry3):
                facc, cv = carry3
                a = None
                b = None
                for u in range(4):
                    a = prod(cv + 2 * u) if a is None else a + prod(cv + 2 * u)
                    b = (prod(cv + 2 * u + 1) if b is None
                         else b + prod(cv + 2 * u + 1))
                ai = plsc.bitcast(a + b, jnp.int32)
                lo = plsc.bitcast(ai << 16, jnp.float32)
                hi = plsc.bitcast(ai & jnp.int32(-65536), jnp.float32)
                return (facc + (lo + hi), cv + 8)

            facc, _ = lax.fori_loop(0, _HW // 8, w_body, (zf, col), unroll=2)
            out_v[pl.ds(off + j * _L, _L)] = facc
            return carry2

        lax.fori_loop(0, _C // _L, blk_body, 0)

    issue(0, 0)

    def pair_body(i, carry):
        c0 = 2 * i
        issue(c0 + 1, 1)
        drain(c0, 0)
        compute(c0, 0)

        @pl.when(c0 + 2 < _NCHUNK)
        def _():
            issue(c0 + 2, 0)

        drain(c0 + 1, 1)
        compute(c0 + 1, 1)
        return carry

    lax.fori_loop(0, _NCHUNK // 2, pair_body, 0)
    drain(_NCHUNK - 1, 0)
    compute(_NCHUNK - 1, 0)
    pltpu.sync_copy(out_v, out_hbm.at[pl.ds(base, _EPW)])


_edge_kernel = functools.partial(
    pl.kernel,
    out_type=jax.ShapeDtypeStruct((_N_EDGES,), jnp.float32),
    mesh=plsc.VectorSubcoreMesh(
        core_axis_name="c", subcore_axis_name="s",
        num_cores=_NC, num_subcores=_NS),
    compiler_params=pltpu.CompilerParams(
        needs_layout_passes=False, use_tc_tiling_on_sc=False),
    scratch_types=[
        pltpu.VMEM((_EPW,), jnp.int32),
        pltpu.VMEM((_EPW,), jnp.int32),
        pltpu.VMEM((_EPW,), jnp.int32),
        pltpu.VMEM((_EPW,), jnp.int32),
        pltpu.VMEM((_C, _HW), jnp.int32),
        pltpu.VMEM((_C, _HW), jnp.int32),
        pltpu.VMEM((_C, _HW), jnp.int32),
        pltpu.VMEM((_C, _HW), jnp.int32),
        pltpu.VMEM((_N_REL, _HW), jnp.int32),
        pltpu.VMEM((_N_SEC, _HW), jnp.int32),
        pltpu.VMEM((_EPW,), jnp.float32),
        pltpu.SemaphoreType.DMA,
        pltpu.SemaphoreType.DMA,
    ],
)(_edge_body)


def kernel(node_repr, head, rel, tail, sector, entity_type_id,
           rel_emb_W, sector_emb_W, type_emb_W):
    etype2d = entity_type_id.astype(jnp.int32).reshape(_N_NODES, 1)
    aug, relp, secp = pl.pallas_call(
        _prep_body,
        out_shape=(
            jax.ShapeDtypeStruct((_N_NODES, _HW), jnp.int32),
            jax.ShapeDtypeStruct((_N_REL, _HW), jnp.int32),
            jax.ShapeDtypeStruct((_N_SEC, _HW), jnp.int32),
        ),
    )(node_repr, etype2d, type_emb_W, rel_emb_W, sector_emb_W)
    return _edge_kernel(
        aug,
        head.astype(jnp.int32), tail.astype(jnp.int32),
        rel.astype(jnp.int32), sector.astype(jnp.int32),
        relp, secp)


# speedup vs baseline: 3.5769x; 3.5769x over previous
import functools

import jax
import jax.numpy as jnp
from jax import lax
from jax.experimental import pallas as pl
from jax.experimental.pallas import tpu as pltpu
from jax.experimental.pallas import tpu_sc as plsc

_N_NODES = 10000
_N_EDGES = 320000
_HIDDEN = 128
_HW = _HIDDEN // 2
_N_REL = 100
_N_SEC = 50

_NC = 2
_NS = 16
_NW = _NC * _NS
_L = 16

_EPW = _N_EDGES // _NW
_C = 80
_NCHUNK = _EPW // _C


def _pack2d(x):
    b = x.astype(jnp.bfloat16)
    lo = lax.bitcast_convert_type(b[:, :_HW], jnp.uint16).astype(jnp.uint32)
    hi = lax.bitcast_convert_type(b[:, _HW:], jnp.uint16).astype(jnp.uint32)
    return lax.bitcast_convert_type(lo | (hi << 16), jnp.int32)


def _prep_body(node_ref, etype_ref, typew_ref, relw_ref, secw_ref,
               aug_ref, relp_ref, secp_ref):
    et = etype_ref[...]
    k = lax.broadcasted_iota(jnp.int32, (et.shape[0], typew_ref.shape[0]), 1)
    onehot = (et == k).astype(jnp.float32)
    aug_ref[...] = _pack2d(node_ref[...] + jnp.dot(
        onehot, typew_ref[...], preferred_element_type=jnp.float32))
    relp_ref[...] = _pack2d(relw_ref[...])
    secp_ref[...] = _pack2d(secw_ref[...])


def _edge_body(a_hbm, head_hbm, tail_hbm, rel_hbm, sec_hbm, relw_hbm,
               secw_hbm, out_hbm, head_v, tail_v, rel_v, sec_v,
               hbuf0, hbuf1, tbuf0, tbuf1, relt, sect, out_v, sem0, sem1):
    wid = lax.axis_index("s") * _NC + lax.axis_index("c")
    base = wid * _EPW
    pltpu.sync_copy(head_hbm.at[pl.ds(base, _EPW)], head_v)
    pltpu.sync_copy(tail_hbm.at[pl.ds(base, _EPW)], tail_v)
    pltpu.sync_copy(rel_hbm.at[pl.ds(base, _EPW)], rel_v)
    pltpu.sync_copy(sec_hbm.at[pl.ds(base, _EPW)], sec_v)
    pltpu.sync_copy(relw_hbm, relt)
    pltpu.sync_copy(secw_hbm, sect)

    bufs = ((hbuf0, tbuf0, sem0), (hbuf1, tbuf1, sem1))

    def issue(c, slot):
        hb, tb, sm = bufs[slot]
        off = c * _C
        pltpu.async_copy(a_hbm.at[head_v.at[pl.ds(off, _C)]], hb, sm)
        pltpu.async_copy(a_hbm.at[tail_v.at[pl.ds(off, _C)]], tb, sm)

    def drain(c, slot):
        hb, tb, sm = bufs[slot]
        off = c * _C
        pltpu.make_async_copy(a_hbm.at[head_v.at[pl.ds(off, _C)]],
                              hb, sm).wait()
        pltpu.make_async_copy(a_hbm.at[tail_v.at[pl.ds(off, _C)]],
                              tb, sm).wait()

    lanes = lax.iota(jnp.int32, _L)

    def _combine(a, b, d):
        m = (lanes & d) == 0
        pa = jnp.take(a, lanes ^ d)
        pb = jnp.take(b, lanes ^ d)
        return jnp.where(m, a, pb) + jnp.where(m, pa, b)

    def compute(c, slot):
        hb2, tb2, _ = bufs[slot]
        off = c * _C

        def blk_body(j, carry2):
            relids = rel_v[pl.ds(off + j * _L, _L)]
            secids = sec_v[pl.ds(off + j * _L, _L)]
            accs = []
            for l in range(_L):
                e = j * _L + l
                rid = relids[l]
                sid = secids[l]
                accb = None
                for g in range(_HW // _L):
                    sl = pl.ds(g * _L, _L)
                    hb = plsc.bitcast(hb2[e, sl], jnp.bfloat16)
                    tb = plsc.bitcast(tb2[e, sl], jnp.bfloat16)
                    rb = plsc.bitcast(relt[rid, sl], jnp.bfloat16)
                    sb = plsc.bitcast(sect[sid, sl], jnp.bfloat16)
                    p = hb * tb * (rb + sb)
                    accb = p if accb is None else accb + p
                ai = plsc.bitcast(accb, jnp.int32)
                lo = plsc.bitcast(ai << 16, jnp.float32)
                hi = plsc.bitcast(ai & jnp.int32(-65536), jnp.float32)
                accs.append(lo + hi)
            d = 1
            while len(accs) > 1:
                accs = [_combine(accs[k], accs[k + 1], d)
                        for k in range(0, len(accs), 2)]
                d *= 2
            out_v[pl.ds(off + j * _L, _L)] = accs[0]
            return carry2

        lax.fori_loop(0, _C // _L, blk_body, 0)

    issue(0, 0)

    def pair_body(i, carry):
        c0 = 2 * i
        issue(c0 + 1, 1)
        drain(c0, 0)
        compute(c0, 0)

        @pl.when(c0 + 2 < _NCHUNK)
        def _():
            issue(c0 + 2, 0)

        drain(c0 + 1, 1)
        compute(c0 + 1, 1)
        return carry

    lax.fori_loop(0, _NCHUNK // 2, pair_body, 0)
    drain(_NCHUNK - 1, 0)
    compute(_NCHUNK - 1, 0)
    pltpu.sync_copy(out_v, out_hbm.at[pl.ds(base, _EPW)])


_edge_kernel = functools.partial(
    pl.kernel,
    out_type=jax.ShapeDtypeStruct((_N_EDGES,), jnp.float32),
    mesh=plsc.VectorSubcoreMesh(
        core_axis_name="c", subcore_axis_name="s",
        num_cores=_NC, num_subcores=_NS),
    compiler_params=pltpu.CompilerParams(
        needs_layout_passes=False, use_tc_tiling_on_sc=False),
    scratch_types=[
        pltpu.VMEM((_EPW,), jnp.int32),
        pltpu.VMEM((_EPW,), jnp.int32),
        pltpu.VMEM((_EPW,), jnp.int32),
        pltpu.VMEM((_EPW,), jnp.int32),
        pltpu.VMEM((_C, _HW), jnp.int32),
        pltpu.VMEM((_C, _HW), jnp.int32),
        pltpu.VMEM((_C, _HW), jnp.int32),
        pltpu.VMEM((_C, _HW), jnp.int32),
        pltpu.VMEM((_N_REL, _HW), jnp.int32),
        pltpu.VMEM((_N_SEC, _HW), jnp.int32),
        pltpu.VMEM((_EPW,), jnp.float32),
        pltpu.SemaphoreType.DMA,
        pltpu.SemaphoreType.DMA,
    ],
)(_edge_body)


def kernel(node_repr, head, rel, tail, sector, entity_type_id,
           rel_emb_W, sector_emb_W, type_emb_W):
    etype2d = entity_type_id.astype(jnp.int32).reshape(_N_NODES, 1)
    aug, relp, secp = pl.pallas_call(
        _prep_body,
        out_shape=(
            jax.ShapeDtypeStruct((_N_NODES, _HW), jnp.int32),
            jax.ShapeDtypeStruct((_N_REL, _HW), jnp.int32),
            jax.ShapeDtypeStruct((_N_SEC, _HW), jnp.int32),
        ),
    )(node_repr, etype2d, type_emb_W, rel_emb_W, sector_emb_W)
    return _edge_kernel(
        aug,
        head.astype(jnp.int32), tail.astype(jnp.int32),
        rel.astype(jnp.int32), sector.astype(jnp.int32),
        relp, secp)
